# baseline (device time: 85150 ns/iter reference)
import jax
import jax.numpy as jnp
from jax import lax
from jax.experimental import pallas as pl
from jax.experimental.pallas import tpu as pltpu

N_DEV = 32
B, SQ, D = 4, 256, 1024
HQ, DH = 8, 128
SCALE = 0.08838834764831843
ROWS = B * SQ
CHUNK = ROWS // N_DEV
N_PEER = N_DEV - 1


def kernel(x, Wq, Wo, Wk, Wv):
    def body(x_ref, wq_ref, wo_ref, wk_ref, wv_ref, out_ref,
             acc_ref, attn_ref, accb_ref, rs_buf, ag_buf, red_ref,
             rs_send, rs_recv, ag_send, ag_recv):
        my = lax.axis_index("i")

        barrier_sem = pltpu.get_barrier_semaphore()
        for j in range(1, N_DEV):
            peer = jnp.mod(my + j, N_DEV)
            pl.semaphore_signal(barrier_sem, inc=1, device_id=(peer,),
                                device_id_type=pl.DeviceIdType.MESH)
        pl.semaphore_wait(barrier_sem, N_PEER)

        xm = x_ref[...].reshape(ROWS, D)
        q = jnp.dot(xm, wq_ref[...], preferred_element_type=jnp.float32)
        k = jnp.dot(xm, wk_ref[...], preferred_element_type=jnp.float32)
        v = jnp.dot(xm, wv_ref[...], preferred_element_type=jnp.float32)
        for b in range(B):
            rows = slice(b * SQ, (b + 1) * SQ)
            for h in range(HQ):
                cols = slice(h * DH, (h + 1) * DH)
                qb = q[rows, cols]
                kb = k[rows, cols]
                vb = v[rows, cols]
                s = lax.dot_general(
                    qb, kb, (((1,), (1,)), ((), ())),
                    preferred_element_type=jnp.float32) * SCALE
                m = jnp.max(s, axis=1, keepdims=True)
                p = jnp.exp(s - m)
                l = jnp.sum(p, axis=1, keepdims=True)
                o = jnp.dot(p, vb, preferred_element_type=jnp.float32) / l
                attn_ref[rows, cols] = o
        acc = jnp.dot(attn_ref[...], wo_ref[...],
                      preferred_element_type=jnp.float32)
        acc_ref[...] = acc
        accb_ref[...] = acc.astype(jnp.bfloat16)

        rs_rdmas = []
        for j in range(1, N_DEV):
            target = jnp.mod(my + j, N_DEV)
            slot = N_DEV - 1 - j
            rdma = pltpu.make_async_remote_copy(
                src_ref=accb_ref.at[pl.ds(target * CHUNK, CHUNK), :],
                dst_ref=rs_buf.at[slot],
                send_sem=rs_send.at[slot],
                recv_sem=rs_recv.at[slot],
                device_id=(target,),
                device_id_type=pl.DeviceIdType.MESH,
            )
            rdma.start()
            rs_rdmas.append(rdma)

        red = acc_ref[pl.ds(my * CHUNK, CHUNK), :]
        for j in range(1, N_DEV):
            slot = N_DEV - 1 - j
            rs_rdmas[j - 1].wait_recv()
            red = red + rs_buf[slot].astype(jnp.float32)
        red_ref[...] = red.astype(jnp.bfloat16)
        out_ref[pl.ds(my * CHUNK, CHUNK), :] = red

        ag_rdmas = []
        for j in range(1, N_DEV):
            target = jnp.mod(my + j, N_DEV)
            slot = N_DEV - 1 - j
            rdma = pltpu.make_async_remote_copy(
                src_ref=red_ref,
                dst_ref=ag_buf.at[slot],
                send_sem=ag_send.at[slot],
                recv_sem=ag_recv.at[slot],
                device_id=(target,),
                device_id_type=pl.DeviceIdType.MESH,
            )
            rdma.start()
            ag_rdmas.append(rdma)

        for j in range(1, N_DEV):
            slot = N_DEV - 1 - j
            sender = jnp.mod(my - j, N_DEV)
            ag_rdmas[j - 1].wait_recv()
            out_ref[pl.ds(sender * CHUNK, CHUNK), :] = (
                ag_buf[slot].astype(jnp.float32))
            rs_rdmas[j - 1].wait_send()
            ag_rdmas[j - 1].wait_send()

    out_flat = pl.pallas_call(
        body,
        out_shape=jax.ShapeDtypeStruct((ROWS, D), jnp.float32),
        in_specs=[pl.BlockSpec(memory_space=pltpu.VMEM)] * 5,
        out_specs=pl.BlockSpec(memory_space=pltpu.VMEM),
        scratch_shapes=[
            pltpu.VMEM((ROWS, D), jnp.float32),
            pltpu.VMEM((ROWS, D), jnp.float32),
            pltpu.VMEM((ROWS, D), jnp.bfloat16),
            pltpu.VMEM((N_PEER, CHUNK, D), jnp.bfloat16),
            pltpu.VMEM((N_PEER, CHUNK, D), jnp.bfloat16),
            pltpu.VMEM((CHUNK, D), jnp.bfloat16),
            pltpu.SemaphoreType.DMA((N_PEER,)),
            pltpu.SemaphoreType.DMA((N_PEER,)),
            pltpu.SemaphoreType.DMA((N_PEER,)),
            pltpu.SemaphoreType.DMA((N_PEER,)),
        ],
        compiler_params=pltpu.CompilerParams(collective_id=0),
    )(x, Wq, Wo, Wk, Wv)
    return out_flat.reshape(B, SQ, D)
